# Initial kernel scaffold; baseline (speedup 1.0000x reference)
import jax
import jax.numpy as jnp
from jax.experimental import pallas as pl


def kernel(x, position, pe):
    raise NotImplementedError("write your pallas kernel here")


# speedup vs baseline: 1.3249x; 1.3249x over previous
import functools

import jax
import jax.numpy as jnp
from jax import lax
from jax.experimental import pallas as pl
from jax.experimental.pallas import tpu as pltpu
from jax.experimental.pallas import tpu_sc as plsc

LANES = 16


@functools.lru_cache(maxsize=None)
def _build(n_rows, d_model, n_cores, n_subcores, chunk):
    n_workers = n_cores * n_subcores
    rows_per_worker = n_rows // n_workers
    n_chunks = rows_per_worker // chunk
    n_slices = d_model // LANES

    mesh = plsc.VectorSubcoreMesh(core_axis_name="c", subcore_axis_name="s")

    @functools.partial(
        pl.kernel,
        mesh=mesh,
        out_type=jax.ShapeDtypeStruct((n_rows, d_model), jnp.float32),
        scratch_types=[
            pltpu.VMEM((chunk,), jnp.int32),
            pltpu.VMEM((chunk, d_model), jnp.float32),
            pltpu.VMEM((chunk, d_model), jnp.float32),
            pltpu.SemaphoreType.DMA,
            pltpu.SemaphoreType.DMA,
        ],
    )
    def k(x_hbm, pos_hbm, pe_hbm, out_hbm, idx_v, pe_v, x_v, sem_g, sem_x):
        wid = lax.axis_index("s") * n_cores + lax.axis_index("c")
        base0 = wid * rows_per_worker

        def do_chunk(g, carry):
            base = base0 + g * chunk
            pltpu.sync_copy(pos_hbm.at[pl.ds(base, chunk)], idx_v)
            gcp = pltpu.async_copy(pe_hbm.at[idx_v], pe_v, sem_g)
            xcp = pltpu.async_copy(x_hbm.at[pl.ds(base, chunk)], x_v, sem_x)
            gcp.wait()
            xcp.wait()

            def add_row(r, c2):
                for j in range(n_slices):
                    sl = pl.ds(j * LANES, LANES)
                    x_v[r, sl] = x_v[r, sl] + pe_v[r, sl]
                return c2

            lax.fori_loop(0, chunk, add_row, 0)
            pltpu.sync_copy(x_v, out_hbm.at[pl.ds(base, chunk)])
            return carry

        lax.fori_loop(0, n_chunks, do_chunk, 0)

    return k


def kernel(x, position, pe):
    b, s, d = x.shape
    n = b * s
    info = plsc.get_sparse_core_info()
    k = _build(n, d, info.num_cores, info.num_subcores, 64)
    out = k(x.reshape(n, d), position.reshape(n), pe)
    return out.reshape(b, s, d)


# speedup vs baseline: 1.8106x; 1.3666x over previous
import functools

import jax
import jax.numpy as jnp
from jax import lax
from jax.experimental import pallas as pl
from jax.experimental.pallas import tpu as pltpu
from jax.experimental.pallas import tpu_sc as plsc

LANES = 16
NBUF = 4


@functools.lru_cache(maxsize=None)
def _build(n_rows, d_model, n_cores, n_subcores, chunk):
    n_workers = n_cores * n_subcores
    rows_per_worker = n_rows // n_workers
    n_chunks = rows_per_worker // chunk
    n_slices = d_model // LANES
    n_mid_rounds = n_chunks // NBUF - 2
    assert n_chunks % NBUF == 0 and n_mid_rounds >= 0

    mesh = plsc.VectorSubcoreMesh(core_axis_name="c", subcore_axis_name="s")

    scratch = (
        [pltpu.VMEM((chunk,), jnp.int32) for _ in range(NBUF)]
        + [pltpu.VMEM((chunk, d_model), jnp.float32) for _ in range(NBUF)]
        + [pltpu.VMEM((chunk, d_model), jnp.float32) for _ in range(NBUF)]
        + [pltpu.SemaphoreType.DMA for _ in range(3 * NBUF + 1)]
    )

    @functools.partial(
        pl.kernel,
        mesh=mesh,
        out_type=jax.ShapeDtypeStruct((n_rows, d_model), jnp.float32),
        scratch_types=scratch,
    )
    def k(x_hbm, pos_hbm, pe_hbm, out_hbm, *scr):
        idx_v = scr[0:NBUF]
        pe_v = scr[NBUF : 2 * NBUF]
        x_v = scr[2 * NBUF : 3 * NBUF]
        sem_g = scr[3 * NBUF : 4 * NBUF]
        sem_x = scr[4 * NBUF : 5 * NBUF]
        sem_w = scr[5 * NBUF : 6 * NBUF]
        sem_i = scr[6 * NBUF]

        wid = lax.axis_index("s") * n_cores + lax.axis_index("c")
        base0 = wid * rows_per_worker

        def issue(g, b):
            base = base0 + g * chunk
            icp = pltpu.async_copy(pos_hbm.at[pl.ds(base, chunk)], idx_v[b], sem_i)
            pltpu.async_copy(x_hbm.at[pl.ds(base, chunk)], x_v[b], sem_x[b])
            icp.wait()
            pltpu.async_copy(pe_hbm.at[idx_v[b]], pe_v[b], sem_g[b])

        def wb_wait(g, b):
            base = base0 + g * chunk
            pltpu.make_async_copy(
                x_v[b], out_hbm.at[pl.ds(base, chunk)], sem_w[b]
            ).wait()

        def finish(g, b):
            base = base0 + g * chunk
            pltpu.make_async_copy(pe_hbm.at[idx_v[b]], pe_v[b], sem_g[b]).wait()
            pltpu.make_async_copy(
                x_hbm.at[pl.ds(base, chunk)], x_v[b], sem_x[b]
            ).wait()

            def add_row(r, c):
                for j in range(n_slices):
                    sl = pl.ds(j * LANES, LANES)
                    x_v[b][r, sl] = x_v[b][r, sl] + pe_v[b][r, sl]
                return c

            lax.fori_loop(0, chunk, add_row, 0)
            pltpu.async_copy(x_v[b], out_hbm.at[pl.ds(base, chunk)], sem_w[b])

        issue(0, 0)
        issue(1, 1)

        for b in range(NBUF):
            finish(b, b)
            b2 = (b + 2) % NBUF
            if b >= 2:
                wb_wait(b - 2, b2)
            issue(b + 2, b2)

        def round_body(r, c):
            g0 = r * NBUF
            for b in range(NBUF):
                g = g0 + b
                b2 = (b + 2) % NBUF
                finish(g, b)
                wb_wait(g - 2, b2)
                issue(g + 2, b2)
            return c

        lax.fori_loop(1, 1 + n_mid_rounds, round_body, 0)

        g0 = n_chunks - NBUF
        for b in range(NBUF):
            g = g0 + b
            b2 = (b + 2) % NBUF
            finish(g, b)
            wb_wait(g - 2, b2)
            if g + 2 < n_chunks:
                issue(g + 2, b2)

        wb_wait(n_chunks - 2, (n_chunks - 2) % NBUF)
        wb_wait(n_chunks - 1, (n_chunks - 1) % NBUF)

    return k


def kernel(x, position, pe):
    b, s, d = x.shape
    n = b * s
    info = plsc.get_sparse_core_info()
    k = _build(n, d, info.num_cores, info.num_subcores, 16)
    out = k(x.reshape(n, d), position.reshape(n), pe)
    return out.reshape(b, s, d)


# speedup vs baseline: 1.8456x; 1.0193x over previous
import functools

import jax
import jax.numpy as jnp
from jax import lax
from jax.experimental import pallas as pl
from jax.experimental.pallas import tpu as pltpu
from jax.experimental.pallas import tpu_sc as plsc

LANES = 16
NBUF = 4


@functools.lru_cache(maxsize=None)
def _build(n_rows, d_model, n_cores, n_subcores, chunk):
    n_workers = n_cores * n_subcores
    rows_per_worker = n_rows // n_workers
    n_chunks = rows_per_worker // chunk
    n_slices = d_model // LANES
    n_mid_rounds = n_chunks // NBUF - 2
    assert n_chunks % NBUF == 0 and n_mid_rounds >= 0

    mesh = plsc.VectorSubcoreMesh(core_axis_name="c", subcore_axis_name="s")

    scratch = (
        [pltpu.VMEM((rows_per_worker,), jnp.int32)]
        + [pltpu.VMEM((chunk, d_model), jnp.float32) for _ in range(NBUF)]
        + [pltpu.VMEM((chunk, d_model), jnp.float32) for _ in range(NBUF)]
        + [pltpu.SemaphoreType.DMA for _ in range(3 * NBUF)]
    )

    @functools.partial(
        pl.kernel,
        mesh=mesh,
        out_type=jax.ShapeDtypeStruct((n_rows, d_model), jnp.float32),
        scratch_types=scratch,
    )
    def k(x_hbm, pos_hbm, pe_hbm, out_hbm, *scr):
        idx_all = scr[0]
        pe_v = scr[1 : 1 + NBUF]
        x_v = scr[1 + NBUF : 1 + 2 * NBUF]
        sem_g = scr[1 + 2 * NBUF : 1 + 3 * NBUF]
        sem_x = scr[1 + 3 * NBUF : 1 + 4 * NBUF]
        sem_w = scr[1 + 4 * NBUF : 1 + 5 * NBUF]

        wid = lax.axis_index("s") * n_cores + lax.axis_index("c")
        base0 = wid * rows_per_worker

        pltpu.sync_copy(pos_hbm.at[pl.ds(base0, rows_per_worker)], idx_all)

        def issue(g, b):
            base = base0 + g * chunk
            iv = idx_all[pl.ds(g * chunk, chunk)]
            pltpu.async_copy(pe_hbm.at[iv], pe_v[b], sem_g[b])
            pltpu.async_copy(x_hbm.at[pl.ds(base, chunk)], x_v[b], sem_x[b])

        def wb_wait(g, b):
            base = base0 + g * chunk
            pltpu.make_async_copy(
                x_v[b], out_hbm.at[pl.ds(base, chunk)], sem_w[b]
            ).wait()

        def finish(g, b):
            base = base0 + g * chunk
            iv = idx_all[pl.ds(g * chunk, chunk)]
            pltpu.make_async_copy(pe_hbm.at[iv], pe_v[b], sem_g[b]).wait()
            pltpu.make_async_copy(
                x_hbm.at[pl.ds(base, chunk)], x_v[b], sem_x[b]
            ).wait()

            def add_row(r, c):
                for j in range(n_slices):
                    sl = pl.ds(j * LANES, LANES)
                    plsc.addupdate(x_v[b].at[r, sl], pe_v[b][r, sl])
                return c

            lax.fori_loop(0, chunk, add_row, 0)
            pltpu.async_copy(x_v[b], out_hbm.at[pl.ds(base, chunk)], sem_w[b])

        issue(0, 0)
        issue(1, 1)

        for b in range(NBUF):
            finish(b, b)
            b2 = (b + 2) % NBUF
            if b >= 2:
                wb_wait(b - 2, b2)
            issue(b + 2, b2)

        def round_body(r, c):
            g0 = r * NBUF
            for b in range(NBUF):
                g = g0 + b
                b2 = (b + 2) % NBUF
                finish(g, b)
                wb_wait(g - 2, b2)
                issue(g + 2, b2)
            return c

        lax.fori_loop(1, 1 + n_mid_rounds, round_body, 0)

        g0 = n_chunks - NBUF
        for b in range(NBUF):
            g = g0 + b
            b2 = (b + 2) % NBUF
            finish(g, b)
            wb_wait(g - 2, b2)
            if g + 2 < n_chunks:
                issue(g + 2, b2)

        wb_wait(n_chunks - 2, (n_chunks - 2) % NBUF)
        wb_wait(n_chunks - 1, (n_chunks - 1) % NBUF)

    return k


def kernel(x, position, pe):
    b, s, d = x.shape
    n = b * s
    info = plsc.get_sparse_core_info()
    k = _build(n, d, info.num_cores, info.num_subcores, 16)
    out = k(x.reshape(n, d), position.reshape(n), pe)
    return out.reshape(b, s, d)
